# baseline (speedup 1.0000x reference)
import functools

import jax
import jax.numpy as jnp
from jax import lax
from jax.experimental import pallas as pl
from jax.experimental.pallas import tpu as pltpu
from jax.experimental.pallas import tpu_sc as plsc

_N = 10000
_DIN = 128
_DOUT = 160
_NS = 5
_DC = _DOUT // _NS
_E = 100000

_NSUB = 16
_CHUNK = 512
_KIDX = _CHUNK // 128
_BPT = 14
_EPT = _BPT * _CHUNK
_EP = _EPT * _NSUB
_KPT = _BPT * _KIDX
_PASS_LVLS = (3, 2)
_ACC_ROWS = max(_PASS_LVLS) * _N
_SEG = _N // _NSUB


def _mm_body(u_ref, it_ref, w_ref, o_ref):
    d = pl.program_id(0)
    x = jnp.where(d == 0, it_ref[...], u_ref[...])
    o_ref[...] = jnp.dot(x, w_ref[0],
                         preferred_element_type=jnp.float32)


def _tables(u, it, w):
    return pl.pallas_call(
        _mm_body,
        grid=(2, _NS),
        in_specs=[
            pl.BlockSpec((_N, _DIN), lambda d, i: (0, 0)),
            pl.BlockSpec((_N, _DIN), lambda d, i: (0, 0)),
            pl.BlockSpec((1, _DIN, _DC), lambda d, i: (i, 0, 0)),
        ],
        out_specs=pl.BlockSpec(
            (_N, _DC), lambda d, i: (d * _NS + i, 0)),
        out_shape=jax.ShapeDtypeStruct((2 * _N * _NS, _DC), jnp.float32),
    )(u, it, w.reshape(_DIN, _NS, _DC).transpose(1, 0, 2))


def _prep_body(r_ref, c_ref, v_ref, gi_ref, si_ref, va_ref):
    d = pl.program_id(0)
    ii = lax.broadcasted_iota(jnp.int32, (_NS, _EP - _E), 1) % _N
    r = jnp.concatenate([r_ref[...], ii], axis=1)
    c = jnp.concatenate([c_ref[...], ii], axis=1)
    lvl = lax.broadcasted_iota(jnp.int32, (_NS, _EP), 0)
    first = jnp.where(d == 0, c, r)
    second = jnp.where(d == 0, r, c)
    gi = d * (_N * _NS) + lvl * _N + first
    lvl_local = jnp.where(lvl < _PASS_LVLS[0], lvl, lvl - _PASS_LVLS[0])
    si = lvl_local * _N + second
    gi_ref[...] = gi.reshape(_NS, _EP // 128, 128)[None]
    si_ref[...] = si.reshape(_NS, _EP // 128, 128)[None]
    va_ref[...] = jnp.concatenate(
        [v_ref[...], jnp.zeros((_NS, _EP - _E), jnp.float32)], axis=1)


def _prep(r, c, v):
    idx_shape = jax.ShapeDtypeStruct((2, _NS, _EP // 128, 128), jnp.int32)
    return pl.pallas_call(
        _prep_body,
        grid=(2,),
        in_specs=[
            pl.BlockSpec((_NS, _E), lambda d: (0, 0)),
            pl.BlockSpec((_NS, _E), lambda d: (0, 0)),
            pl.BlockSpec((_NS, _E), lambda d: (0, 0)),
        ],
        out_specs=[
            pl.BlockSpec((1, _NS, _EP // 128, 128), lambda d: (d, 0, 0, 0)),
            pl.BlockSpec((1, _NS, _EP // 128, 128), lambda d: (d, 0, 0, 0)),
            pl.BlockSpec((_NS, _EP), lambda d: (0, 0)),
        ],
        out_shape=[
            idx_shape,
            idx_shape,
            jax.ShapeDtypeStruct((_NS, _EP), jnp.float32),
        ],
    )(r, c, v)


@functools.partial(
    pl.kernel,
    out_type=jax.ShapeDtypeStruct((2, _N, _DOUT), jnp.float32),
    mesh=plsc.VectorSubcoreMesh(core_axis_name="c", subcore_axis_name="s"),
    compiler_params=pltpu.CompilerParams(
        use_tc_tiling_on_sc=False, needs_layout_passes=False),
    scratch_types=[
        pltpu.VMEM((_KPT, 128), jnp.int32),
        pltpu.VMEM((_KPT, 128), jnp.int32),
        pltpu.VMEM((_EPT,), jnp.float32),
        pltpu.VMEM((3 * _CHUNK, _DC), jnp.float32),
        pltpu.VMEM_SHARED((_ACC_ROWS, _DC), jnp.float32),
        pltpu.SemaphoreType.DMA,
        pltpu.SemaphoreType.DMA,
    ],
)
def _sc_aggregate(table, gidx, sidx, vals, zeros, out,
                  gi_v, si_v, vv, rows_v, acc, gsem, ssem):
    d = lax.axis_index("c")
    s = lax.axis_index("s")

    def issue_gather(q, buf):
        for j in range(_KIDX):
            pltpu.async_copy(
                table.at[gi_v.at[q * _KIDX + j]],
                rows_v.at[pl.ds(buf * _CHUNK + j * 128, 128)],
                gsem,
            )

    def wait_gather(buf):
        for j in range(_KIDX):
            pltpu.make_async_copy(
                table.at[gi_v.at[j]],
                rows_v.at[pl.ds(buf * _CHUNK + j * 128, 128)],
                gsem,
            ).wait()

    def issue_scatter(q, buf):
        for j in range(_KIDX):
            pltpu.async_copy(
                rows_v.at[pl.ds(buf * _CHUNK + j * 128, 128)],
                acc.at[si_v.at[q * _KIDX + j]],
                ssem,
                add=True,
            )

    def wait_scatter(buf):
        for j in range(_KIDX):
            pltpu.make_async_copy(
                rows_v.at[pl.ds(buf * _CHUNK + j * 128, 128)],
                acc.at[si_v.at[j]],
                ssem,
            ).wait()

    def scale(q, buf):
        @plsc.parallel_loop(0, _CHUNK // 16, 1)
        def g_body(g):
            vv16 = vv[pl.ds(q * _CHUNK + g * 16, 16)]
            rowi = (buf * _CHUNK + g * 16
                    + lax.iota(jnp.int32, 16))
            for j in range(_DC):
                colj = jnp.full((16,), j, jnp.int32)
                x = plsc.load_gather(rows_v, [rowi, colj])
                plsc.store_scatter(rows_v, [rowi, colj], x * vv16)

    base_lvl = 0
    for nlvl in _PASS_LVLS:
        stripe = nlvl * _SEG

        pltpu.sync_copy(zeros.at[pl.ds(0, stripe)],
                        acc.at[pl.ds(s * stripe, stripe)])
        plsc.subcore_barrier()

        def level_body(l, carry, base_lvl=base_lvl):
            i = base_lvl + l
            pltpu.sync_copy(gidx.at[d, i, pl.ds(s * _KPT, _KPT)], gi_v)
            pltpu.sync_copy(sidx.at[d, i, pl.ds(s * _KPT, _KPT)], si_v)
            pltpu.sync_copy(vals.at[i, pl.ds(s * _EPT, _EPT)], vv)

            issue_gather(0, 0)

            def slot_body(q, c):
                bq = lax.rem(q, 3)

                wait_gather(bq)

                @pl.when(q < _BPT - 1)
                def _():
                    issue_gather(q + 1, lax.rem(q + 1, 3))

                scale(q, bq)

                @pl.when(q > 0)
                def _():
                    wait_scatter(lax.rem(q + 2, 3))

                issue_scatter(q, bq)
                return c

            lax.fori_loop(0, _BPT, slot_body, 0)
            wait_scatter(lax.rem(_BPT - 1, 3))
            return carry

        lax.fori_loop(0, nlvl, level_body, 0)

        plsc.subcore_barrier()
        for il in range(nlvl):
            pltpu.sync_copy(
                acc.at[pl.ds(il * _N + s * _SEG, _SEG)],
                out.at[d, pl.ds(s * _SEG, _SEG),
                       pl.ds((base_lvl + il) * _DC, _DC)],
            )
        plsc.subcore_barrier()

        base_lvl += nlvl


def kernel(user_inputs, item_inputs, support_rows, support_cols,
           support_vals, weight):
    table = _tables(user_inputs, item_inputs, weight)
    gidx, sidx, vals = _prep(support_rows, support_cols, support_vals)
    zeros = jnp.zeros((_PASS_LVLS[0] * _SEG, _DC), jnp.float32)
    out = _sc_aggregate(table, gidx, sidx, vals, zeros)
    return (out[0], out[1])


# speedup vs baseline: 2.5752x; 2.5752x over previous
import functools

import jax
import jax.numpy as jnp
from jax import lax
from jax.experimental import pallas as pl
from jax.experimental.pallas import tpu as pltpu
from jax.experimental.pallas import tpu_sc as plsc

_N = 10000
_DIN = 128
_DOUT = 160
_NS = 5
_DC = _DOUT // _NS
_E = 100000

_NSUB = 16
_CHUNK = 512
_KIDX = _CHUNK // 128
_BPT = 14
_EPT = _BPT * _CHUNK
_EP = _EPT * _NSUB
_KPT = _BPT * _KIDX
_PASS_LVLS = (3, 2)
_ACC_ROWS = max(_PASS_LVLS) * _N
_SEG = _N // _NSUB


def _mm_body(u_ref, it_ref, w_ref, o_ref):
    d = pl.program_id(0)
    x = jnp.where(d == 0, it_ref[...], u_ref[...])
    o_ref[...] = jnp.dot(x, w_ref[0],
                         preferred_element_type=jnp.float32)


def _tables(u, it, w):
    return pl.pallas_call(
        _mm_body,
        grid=(2, _NS),
        in_specs=[
            pl.BlockSpec((_N, _DIN), lambda d, i: (0, 0)),
            pl.BlockSpec((_N, _DIN), lambda d, i: (0, 0)),
            pl.BlockSpec((1, _DIN, _DC), lambda d, i: (i, 0, 0)),
        ],
        out_specs=pl.BlockSpec(
            (_N, _DC), lambda d, i: (d * _NS + i, 0)),
        out_shape=jax.ShapeDtypeStruct((2 * _N * _NS, _DC), jnp.float32),
    )(u, it, w.reshape(_DIN, _NS, _DC).transpose(1, 0, 2))


def _prep_body(r_ref, c_ref, v_ref, gi_ref, si_ref, va_ref):
    d = pl.program_id(0)
    ii = lax.broadcasted_iota(jnp.int32, (_NS, _EP - _E), 1) % _N
    r = jnp.concatenate([r_ref[...], ii], axis=1)
    c = jnp.concatenate([c_ref[...], ii], axis=1)
    lvl = lax.broadcasted_iota(jnp.int32, (_NS, _EP), 0)
    first = jnp.where(d == 0, c, r)
    second = jnp.where(d == 0, r, c)
    gi = d * (_N * _NS) + lvl * _N + first
    lvl_local = jnp.where(lvl < _PASS_LVLS[0], lvl, lvl - _PASS_LVLS[0])
    si = lvl_local * _N + second
    gi_ref[...] = gi.reshape(_NS, _EP // 128, 128)[None]
    si_ref[...] = si.reshape(_NS, _EP // 128, 128)[None]
    va_ref[...] = jnp.concatenate(
        [v_ref[...], jnp.zeros((_NS, _EP - _E), jnp.float32)], axis=1)


def _prep(r, c, v):
    idx_shape = jax.ShapeDtypeStruct((2, _NS, _EP // 128, 128), jnp.int32)
    return pl.pallas_call(
        _prep_body,
        grid=(2,),
        in_specs=[
            pl.BlockSpec((_NS, _E), lambda d: (0, 0)),
            pl.BlockSpec((_NS, _E), lambda d: (0, 0)),
            pl.BlockSpec((_NS, _E), lambda d: (0, 0)),
        ],
        out_specs=[
            pl.BlockSpec((1, _NS, _EP // 128, 128), lambda d: (d, 0, 0, 0)),
            pl.BlockSpec((1, _NS, _EP // 128, 128), lambda d: (d, 0, 0, 0)),
            pl.BlockSpec((_NS, _EP), lambda d: (0, 0)),
        ],
        out_shape=[
            idx_shape,
            idx_shape,
            jax.ShapeDtypeStruct((_NS, _EP), jnp.float32),
        ],
    )(r, c, v)


@functools.partial(
    pl.kernel,
    out_type=jax.ShapeDtypeStruct((2, _N, _DOUT), jnp.float32),
    mesh=plsc.VectorSubcoreMesh(core_axis_name="c", subcore_axis_name="s"),
    compiler_params=pltpu.CompilerParams(
        use_tc_tiling_on_sc=False, needs_layout_passes=False),
    scratch_types=[
        pltpu.VMEM((_KPT, 128), jnp.int32),
        pltpu.VMEM((_KPT, 128), jnp.int32),
        pltpu.VMEM((_EPT,), jnp.float32),
        pltpu.VMEM((3 * _CHUNK, _DC), jnp.float32),
        pltpu.VMEM_SHARED((_ACC_ROWS, _DC), jnp.float32),
        pltpu.SemaphoreType.DMA,
        pltpu.SemaphoreType.DMA,
    ],
)
def _sc_aggregate(table, gidx, sidx, vals, zeros, out,
                  gi_v, si_v, vv, rows_v, acc, gsem, ssem):
    d = lax.axis_index("c")
    s = lax.axis_index("s")

    def _rows2d(buf, j):
        return rows_v.at[pl.ds(buf * _CHUNK + j * 128, 128)]

    def issue_gather(q, buf):
        for j in range(_KIDX):
            pltpu.async_copy(
                table.at[gi_v.at[q * _KIDX + j]],
                _rows2d(buf, j),
                gsem,
            )

    def wait_gather(buf):
        for j in range(_KIDX):
            pltpu.make_async_copy(
                table.at[gi_v.at[j]],
                _rows2d(buf, j),
                gsem,
            ).wait()

    def issue_scatter(q, buf):
        for j in range(_KIDX):
            pltpu.async_copy(
                _rows2d(buf, j),
                acc.at[si_v.at[q * _KIDX + j]],
                ssem,
                add=True,
            )

    def wait_scatter(buf):
        for j in range(_KIDX):
            pltpu.make_async_copy(
                _rows2d(buf, j),
                acc.at[si_v.at[j]],
                ssem,
            ).wait()

    def scale(q, buf):
        @plsc.parallel_loop(0, _CHUNK // 16, 1)
        def g_body(g):
            vv16 = vv[pl.ds(q * _CHUNK + g * 16, 16)]
            e0 = buf * _CHUNK + g * 16
            for k in range(16):
                v = vv16[k]
                rows_v[e0 + k, pl.ds(0, 16)] = (
                    rows_v[e0 + k, pl.ds(0, 16)] * v)
                rows_v[e0 + k, pl.ds(16, 16)] = (
                    rows_v[e0 + k, pl.ds(16, 16)] * v)

    base_lvl = 0
    for nlvl in _PASS_LVLS:
        stripe = nlvl * _SEG

        pltpu.sync_copy(zeros.at[pl.ds(0, stripe)],
                        acc.at[pl.ds(s * stripe, stripe)])
        plsc.subcore_barrier()

        def level_body(l, carry, base_lvl=base_lvl):
            i = base_lvl + l
            pltpu.sync_copy(gidx.at[d, i, pl.ds(s * _KPT, _KPT)], gi_v)
            pltpu.sync_copy(sidx.at[d, i, pl.ds(s * _KPT, _KPT)], si_v)
            pltpu.sync_copy(vals.at[i, pl.ds(s * _EPT, _EPT)], vv)

            issue_gather(0, 0)

            def slot_body(q, c):
                bq = lax.rem(q, 3)

                wait_gather(bq)

                @pl.when(q < _BPT - 1)
                def _():
                    issue_gather(q + 1, lax.rem(q + 1, 3))

                scale(q, bq)

                @pl.when(q > 0)
                def _():
                    wait_scatter(lax.rem(q + 2, 3))

                issue_scatter(q, bq)
                return c

            lax.fori_loop(0, _BPT, slot_body, 0)
            wait_scatter(lax.rem(_BPT - 1, 3))
            return carry

        lax.fori_loop(0, nlvl, level_body, 0)

        plsc.subcore_barrier()
        for il in range(nlvl):
            pltpu.sync_copy(
                acc.at[pl.ds(il * _N + s * _SEG, _SEG)],
                out.at[d, pl.ds(s * _SEG, _SEG),
                       pl.ds((base_lvl + il) * _DC, _DC)],
            )
        plsc.subcore_barrier()

        base_lvl += nlvl


def kernel(user_inputs, item_inputs, support_rows, support_cols,
           support_vals, weight):
    table = _tables(user_inputs, item_inputs, weight)
    gidx, sidx, vals = _prep(support_rows, support_cols, support_vals)
    zeros = jnp.zeros((_PASS_LVLS[0] * _SEG, _DC), jnp.float32)
    out = _sc_aggregate(table, gidx, sidx, vals, zeros)
    return (out[0], out[1])


# speedup vs baseline: 3.0113x; 1.1693x over previous
import functools

import jax
import jax.numpy as jnp
from jax import lax
from jax.experimental import pallas as pl
from jax.experimental.pallas import tpu as pltpu
from jax.experimental.pallas import tpu_sc as plsc

_N = 10000
_DIN = 128
_DOUT = 160
_NS = 5
_DC = _DOUT // _NS
_E = 100000

_NSUB = 16
_CHUNK = 512
_KIDX = _CHUNK // 128
_BPT = 14
_EPT = _BPT * _CHUNK
_EP = _EPT * _NSUB
_KPT = _BPT * _KIDX
_PASSES = ((0, 2), (2, 2), (4, 1))
_ACC_ROWS = 2 * _N
_SEG = _N // _NSUB


def _mm_body(u_ref, it_ref, w_ref, oa_ref, ob_ref):
    d = pl.program_id(0)
    x = jnp.where(d == 0, it_ref[...], u_ref[...])
    oa_ref[...] = jnp.dot(x, w_ref[:, pl.ds(0, 4 * _DC)],
                          preferred_element_type=jnp.float32)
    ob_ref[...] = jnp.dot(x, w_ref[:, pl.ds(4 * _DC, _DC)],
                          preferred_element_type=jnp.float32)


def _tables(u, it, w):
    return pl.pallas_call(
        _mm_body,
        grid=(2,),
        in_specs=[
            pl.BlockSpec((_N, _DIN), lambda d: (0, 0)),
            pl.BlockSpec((_N, _DIN), lambda d: (0, 0)),
            pl.BlockSpec((_DIN, _DOUT), lambda d: (0, 0)),
        ],
        out_specs=[
            pl.BlockSpec((_N, 4 * _DC), lambda d: (d, 0)),
            pl.BlockSpec((_N, _DC), lambda d: (d, 0)),
        ],
        out_shape=[
            jax.ShapeDtypeStruct((2 * _N, 4 * _DC), jnp.float32),
            jax.ShapeDtypeStruct((2 * _N, _DC), jnp.float32),
        ],
    )(u, it, w)


def _prep_body(r_ref, c_ref, v_ref, gi_ref, si_ref, va_ref):
    d = pl.program_id(0)
    ii = lax.broadcasted_iota(jnp.int32, (_NS, _EP - _E), 1) % _N
    r = jnp.concatenate([r_ref[...], ii], axis=1)
    c = jnp.concatenate([c_ref[...], ii], axis=1)
    lvl = lax.broadcasted_iota(jnp.int32, (_NS, _EP), 0)
    first = jnp.where(d == 0, c, r)
    second = jnp.where(d == 0, r, c)
    gi = jnp.where(lvl < 4,
                   (d * _N + first) * 4 + lvl,
                   d * _N + first)
    lvl_local = jnp.where(lvl < 2, lvl, jnp.where(lvl < 4, lvl - 2, 0))
    si = lvl_local * _N + second
    gi_ref[...] = gi.reshape(_NS, _EP // 128, 128)[None]
    si_ref[...] = si.reshape(_NS, _EP // 128, 128)[None]
    va_ref[...] = jnp.concatenate(
        [v_ref[...], jnp.zeros((_NS, _EP - _E), jnp.float32)], axis=1)


def _prep(r, c, v):
    idx_shape = jax.ShapeDtypeStruct((2, _NS, _EP // 128, 128), jnp.int32)
    return pl.pallas_call(
        _prep_body,
        grid=(2,),
        in_specs=[
            pl.BlockSpec((_NS, _E), lambda d: (0, 0)),
            pl.BlockSpec((_NS, _E), lambda d: (0, 0)),
            pl.BlockSpec((_NS, _E), lambda d: (0, 0)),
        ],
        out_specs=[
            pl.BlockSpec((1, _NS, _EP // 128, 128), lambda d: (d, 0, 0, 0)),
            pl.BlockSpec((1, _NS, _EP // 128, 128), lambda d: (d, 0, 0, 0)),
            pl.BlockSpec((_NS, _EP), lambda d: (0, 0)),
        ],
        out_shape=[
            idx_shape,
            idx_shape,
            jax.ShapeDtypeStruct((_NS, _EP), jnp.float32),
        ],
    )(r, c, v)


@functools.partial(
    pl.kernel,
    out_type=jax.ShapeDtypeStruct((2, _N, _DOUT), jnp.float32),
    mesh=plsc.VectorSubcoreMesh(core_axis_name="c", subcore_axis_name="s"),
    compiler_params=pltpu.CompilerParams(
        use_tc_tiling_on_sc=False, needs_layout_passes=False),
    scratch_types=[
        pltpu.VMEM((_KPT, 128), jnp.int32),
        pltpu.VMEM((_KPT, 128), jnp.int32),
        pltpu.VMEM((_EPT,), jnp.float32),
        pltpu.VMEM((4 * _CHUNK, _DC), jnp.float32),
        pltpu.VMEM_SHARED((_ACC_ROWS, _DC), jnp.float32),
        pltpu.SemaphoreType.DMA,
        pltpu.SemaphoreType.DMA,
    ],
)
def _sc_aggregate(table_a, table_b, gidx, sidx, vals, zeros, out,
                  gi_v, si_v, vv, rows_v, acc, gsem, ssem):
    d = lax.axis_index("c")
    s = lax.axis_index("s")

    def _rows(buf, j):
        return rows_v.at[pl.ds(buf * _CHUNK + j * 128, 128)]

    def scale(q, buf):
        @plsc.parallel_loop(0, _CHUNK // 16, 1)
        def g_body(g):
            vv16 = vv[pl.ds(q * _CHUNK + g * 16, 16)]
            e0 = buf * _CHUNK + g * 16
            for k in range(16):
                v = vv16[k]
                rows_v[e0 + k, pl.ds(0, 16)] = (
                    rows_v[e0 + k, pl.ds(0, 16)] * v)
                rows_v[e0 + k, pl.ds(16, 16)] = (
                    rows_v[e0 + k, pl.ds(16, 16)] * v)

    def do_level(i, tab):
        pltpu.sync_copy(gidx.at[d, i, pl.ds(s * _KPT, _KPT)], gi_v)
        pltpu.sync_copy(sidx.at[d, i, pl.ds(s * _KPT, _KPT)], si_v)
        pltpu.sync_copy(vals.at[i, pl.ds(s * _EPT, _EPT)], vv)

        def issue_gather(q, buf):
            for j in range(_KIDX):
                pltpu.async_copy(
                    tab.at[gi_v.at[q * _KIDX + j]], _rows(buf, j), gsem)

        def wait_gather(buf):
            for j in range(_KIDX):
                pltpu.make_async_copy(
                    tab.at[gi_v.at[j]], _rows(buf, j), gsem).wait()

        def issue_scatter(q, buf):
            for j in range(_KIDX):
                pltpu.async_copy(
                    _rows(buf, j), acc.at[si_v.at[q * _KIDX + j]],
                    ssem, add=True)

        def wait_scatter(buf):
            for j in range(_KIDX):
                pltpu.make_async_copy(
                    _rows(buf, j), acc.at[si_v.at[j]], ssem).wait()

        issue_gather(0, 0)
        issue_gather(1, 1)

        def slot_body(q, c):
            bq = lax.rem(q, 4)
            wait_gather(bq)

            @pl.when(q >= 2)
            def _():
                wait_scatter(lax.rem(q + 2, 4))

            @pl.when(q + 2 < _BPT)
            def _():
                issue_gather(q + 2, lax.rem(q + 2, 4))

            scale(q, bq)
            issue_scatter(q, bq)
            return c

        lax.fori_loop(0, _BPT, slot_body, 0)
        wait_scatter(lax.rem(_BPT - 2, 4))
        wait_scatter(lax.rem(_BPT - 1, 4))

    for base_lvl, nlvl in _PASSES:
        stripe = nlvl * _SEG

        pltpu.sync_copy(zeros.at[pl.ds(0, stripe)],
                        acc.at[pl.ds(s * stripe, stripe)])
        plsc.subcore_barrier()

        if nlvl == 1:
            do_level(base_lvl, table_b)
        else:
            def level_body(l, carry, base_lvl=base_lvl):
                do_level(base_lvl + l, table_a)
                return carry

            lax.fori_loop(0, nlvl, level_body, 0)

        plsc.subcore_barrier()
        for il in range(nlvl):
            pltpu.sync_copy(
                acc.at[pl.ds(il * _N + s * _SEG, _SEG)],
                out.at[d, pl.ds(s * _SEG, _SEG),
                       pl.ds((base_lvl + il) * _DC, _DC)],
            )
        plsc.subcore_barrier()


def kernel(user_inputs, item_inputs, support_rows, support_cols,
           support_vals, weight):
    table_a, table_b = _tables(user_inputs, item_inputs, weight)
    table_a = table_a.reshape(8 * _N, _DC)
    gidx, sidx, vals = _prep(support_rows, support_cols, support_vals)
    zeros = jnp.zeros((2 * _SEG, _DC), jnp.float32)
    out = _sc_aggregate(table_a, table_b, gidx, sidx, vals, zeros)
    return (out[0], out[1])


# speedup vs baseline: 3.0687x; 1.0191x over previous
import functools

import jax
import jax.numpy as jnp
from jax import lax
from jax.experimental import pallas as pl
from jax.experimental.pallas import tpu as pltpu
from jax.experimental.pallas import tpu_sc as plsc

_N = 10000
_DIN = 128
_DOUT = 160
_NS = 5
_DC = _DOUT // _NS
_E = 100000

_NSUB = 16
_CHUNK = 512
_KIDX = _CHUNK // 128
_BPT = 13
_EPT = _BPT * _CHUNK
_EP = _EPT * _NSUB
_KPT = _BPT * _KIDX
_PASSES = ((0, 2), (2, 2), (4, 1))
_ACC_ROWS = 2 * _N
_SEG = _N // _NSUB


def _mm_body(u_ref, it_ref, w_ref, oa_ref, ob_ref):
    d = pl.program_id(0)
    x = jnp.where(d == 0, it_ref[...], u_ref[...])
    oa_ref[...] = jnp.dot(x, w_ref[:, pl.ds(0, 4 * _DC)],
                          preferred_element_type=jnp.float32
                          ).astype(jnp.bfloat16)
    ob_ref[...] = jnp.dot(x, w_ref[:, pl.ds(4 * _DC, _DC)],
                          preferred_element_type=jnp.float32
                          ).astype(jnp.bfloat16)


def _tables(u, it, w):
    return pl.pallas_call(
        _mm_body,
        grid=(2,),
        in_specs=[
            pl.BlockSpec((_N, _DIN), lambda d: (0, 0)),
            pl.BlockSpec((_N, _DIN), lambda d: (0, 0)),
            pl.BlockSpec((_DIN, _DOUT), lambda d: (0, 0)),
        ],
        out_specs=[
            pl.BlockSpec((_N, 4 * _DC), lambda d: (d, 0)),
            pl.BlockSpec((_N, _DC), lambda d: (d, 0)),
        ],
        out_shape=[
            jax.ShapeDtypeStruct((2 * _N, 4 * _DC), jnp.bfloat16),
            jax.ShapeDtypeStruct((2 * _N, _DC), jnp.bfloat16),
        ],
    )(u, it, w)


def _prep_body(r_ref, c_ref, v_ref, gi_ref, si_ref, va_ref):
    d = pl.program_id(0)
    ii = lax.broadcasted_iota(jnp.int32, (_NS, _EP - _E), 1) % _N
    r = jnp.concatenate([r_ref[...], ii], axis=1)
    c = jnp.concatenate([c_ref[...], ii], axis=1)
    lvl = lax.broadcasted_iota(jnp.int32, (_NS, _EP), 0)
    first = jnp.where(d == 0, c, r)
    second = jnp.where(d == 0, r, c)
    gi = jnp.where(lvl < 4,
                   (d * _N + first) * 4 + lvl,
                   d * _N + first)
    lvl_local = jnp.where(lvl < 2, lvl, jnp.where(lvl < 4, lvl - 2, 0))
    si = lvl_local * _N + second
    gi_ref[...] = gi.reshape(_NS, _EP // 128, 128)[None]
    si_ref[...] = si.reshape(_NS, _EP // 128, 128)[None]
    va_ref[...] = jnp.concatenate(
        [v_ref[...], jnp.zeros((_NS, _EP - _E), jnp.float32)], axis=1)


def _prep(r, c, v):
    idx_shape = jax.ShapeDtypeStruct((2, _NS, _EP // 128, 128), jnp.int32)
    return pl.pallas_call(
        _prep_body,
        grid=(2,),
        in_specs=[
            pl.BlockSpec((_NS, _E), lambda d: (0, 0)),
            pl.BlockSpec((_NS, _E), lambda d: (0, 0)),
            pl.BlockSpec((_NS, _E), lambda d: (0, 0)),
        ],
        out_specs=[
            pl.BlockSpec((1, _NS, _EP // 128, 128), lambda d: (d, 0, 0, 0)),
            pl.BlockSpec((1, _NS, _EP // 128, 128), lambda d: (d, 0, 0, 0)),
            pl.BlockSpec((_NS, _EP), lambda d: (0, 0)),
        ],
        out_shape=[
            idx_shape,
            idx_shape,
            jax.ShapeDtypeStruct((_NS, _EP), jnp.float32),
        ],
    )(r, c, v)


@functools.partial(
    pl.kernel,
    out_type=jax.ShapeDtypeStruct((2, _N, _DOUT), jnp.float32),
    mesh=plsc.VectorSubcoreMesh(core_axis_name="c", subcore_axis_name="s"),
    compiler_params=pltpu.CompilerParams(
        use_tc_tiling_on_sc=False, needs_layout_passes=False),
    scratch_types=[
        pltpu.VMEM((_KPT, 128), jnp.int32),
        pltpu.VMEM((_KPT, 128), jnp.int32),
        pltpu.VMEM((_EPT,), jnp.float32),
        pltpu.VMEM((3 * _CHUNK, _DC), jnp.bfloat16),
        pltpu.VMEM((2 * _CHUNK, _DC), jnp.float32),
        pltpu.VMEM_SHARED((_ACC_ROWS, _DC), jnp.float32),
        pltpu.SemaphoreType.DMA,
        pltpu.SemaphoreType.DMA,
    ],
)
def _sc_aggregate(table_a, table_b, gidx, sidx, vals, zeros, out,
                  gi_v, si_v, vv, rows_bf, rows_f, acc, gsem, ssem):
    d = lax.axis_index("c")
    s = lax.axis_index("s")

    def _bf(buf, j):
        return rows_bf.at[pl.ds(buf * _CHUNK + j * 128, 128)]

    def _f32(buf, j):
        return rows_f.at[pl.ds(buf * _CHUNK + j * 128, 128)]

    def scale(q, bq, fq):
        @plsc.parallel_loop(0, _CHUNK // 16, 1)
        def g_body(g):
            vv16 = vv[pl.ds(q * _CHUNK + g * 16, 16)]
            e0b = bq * _CHUNK + g * 16
            e0f = fq * _CHUNK + g * 16
            for k in range(16):
                v = vv16[k]
                x = rows_bf[e0b + k, pl.ds(0, _DC)]
                lo, hi = plsc.unpack(x, format=plsc.PackFormat.INTERLEAVED)
                rows_f[e0f + k, pl.ds(0, 16)] = lo * v
                rows_f[e0f + k, pl.ds(16, 16)] = hi * v

    def do_level(i, tab):
        pltpu.sync_copy(gidx.at[d, i, pl.ds(s * _KPT, _KPT)], gi_v)
        pltpu.sync_copy(sidx.at[d, i, pl.ds(s * _KPT, _KPT)], si_v)
        pltpu.sync_copy(vals.at[i, pl.ds(s * _EPT, _EPT)], vv)

        def issue_gather(q, buf):
            for j in range(_KIDX):
                pltpu.async_copy(
                    tab.at[gi_v.at[q * _KIDX + j]], _bf(buf, j), gsem)

        def wait_gather(buf):
            for j in range(_KIDX):
                pltpu.make_async_copy(
                    tab.at[gi_v.at[j]], _bf(buf, j), gsem).wait()

        def issue_scatter(q, buf):
            for j in range(_KIDX):
                pltpu.async_copy(
                    _f32(buf, j), acc.at[si_v.at[q * _KIDX + j]],
                    ssem, add=True)

        def wait_scatter(buf):
            for j in range(_KIDX):
                pltpu.make_async_copy(
                    _f32(buf, j), acc.at[si_v.at[j]], ssem).wait()

        issue_gather(0, 0)
        issue_gather(1, 1)

        def slot_body(q, c):
            bq = lax.rem(q, 3)
            fq = lax.rem(q, 2)
            wait_gather(bq)

            @pl.when(q + 2 < _BPT)
            def _():
                issue_gather(q + 2, lax.rem(q + 2, 3))

            scale(q, bq, fq)

            @pl.when(q >= 1)
            def _():
                wait_scatter(lax.rem(q + 1, 2))

            issue_scatter(q, fq)
            return c

        lax.fori_loop(0, _BPT, slot_body, 0)
        wait_scatter(lax.rem(_BPT - 1, 2))

    for base_lvl, nlvl in _PASSES:
        stripe = nlvl * _SEG

        pltpu.sync_copy(zeros.at[pl.ds(0, stripe)],
                        acc.at[pl.ds(s * stripe, stripe)])
        plsc.subcore_barrier()

        if nlvl == 1:
            do_level(base_lvl, table_b)
        else:
            def level_body(l, carry, base_lvl=base_lvl):
                do_level(base_lvl + l, table_a)
                return carry

            lax.fori_loop(0, nlvl, level_body, 0)

        plsc.subcore_barrier()
        for il in range(nlvl):
            pltpu.sync_copy(
                acc.at[pl.ds(il * _N + s * _SEG, _SEG)],
                out.at[d, pl.ds(s * _SEG, _SEG),
                       pl.ds((base_lvl + il) * _DC, _DC)],
            )
        plsc.subcore_barrier()


def kernel(user_inputs, item_inputs, support_rows, support_cols,
           support_vals, weight):
    perm = jnp.arange(_DOUT)
    perm = ((perm // _DC) * _DC + (perm % _DC % 2) * 16 + (perm % _DC) // 2)
    w_perm = weight[:, perm]
    table_a, table_b = _tables(user_inputs, item_inputs, w_perm)
    table_a = table_a.reshape(8 * _N, _DC)
    gidx, sidx, vals = _prep(support_rows, support_cols, support_vals)
    zeros = jnp.zeros((2 * _SEG, _DC), jnp.float32)
    out = _sc_aggregate(table_a, table_b, gidx, sidx, vals, zeros)
    return (out[0], out[1])
